# baseline (speedup 1.0000x reference)
import functools

import jax
import jax.numpy as jnp
from jax import lax
from jax.experimental import pallas as pl
from jax.experimental.pallas import tpu as pltpu
from jax.experimental.pallas import tpu_sc as plsc

_N = 10000
_DH = 128
_E = 320000
_EPS = 1e-5

_NC = 2
_NS = 16
_NW = _NC * _NS
_CH = 128
_EPAD = 327680
_NROW_PAD = 10112
_RPT = _NROW_PAD // _NS
_CPP = 8
_K0 = 19
_K1 = 1
_NB = 2


def _build_sc_agg():
    mesh = plsc.VectorSubcoreMesh(core_axis_name="c", subcore_axis_name="s")

    @functools.partial(
        pl.kernel,
        out_type=jax.ShapeDtypeStruct((_NC, _NROW_PAD, _DH), jnp.float32),
        mesh=mesh,
        scratch_types=[
            pltpu.VMEM((_CPP, _CH), jnp.int32),
            pltpu.VMEM((_CPP, _CH), jnp.int32),
            [pltpu.VMEM((_CH, _DH), jnp.float32) for _ in range(_NB)],
            pltpu.VMEM_SHARED((_NROW_PAD, _DH), jnp.float32),
            [pltpu.SemaphoreType.DMA for _ in range(_NB)],
            [pltpu.SemaphoreType.DMA for _ in range(_NB)],
        ],
    )
    def sc_agg(h_hbm, src_hbm, dst_hbm, zeros_hbm, out_hbm,
               srcv, dstv, rows, agg, gs, ss):
        c = lax.axis_index("c")
        s = lax.axis_index("s")
        w = c * _NS + s
        pltpu.sync_copy(zeros_hbm, rows[0])
        r0 = s * _RPT
        nfull = _RPT // _CH
        for zi in range(nfull):
            pltpu.sync_copy(rows[0], agg.at[pl.ds(r0 + zi * _CH, _CH)])
        rem = _RPT - nfull * _CH
        if rem:
            pltpu.sync_copy(rows[0].at[pl.ds(0, rem)],
                            agg.at[pl.ds(r0 + nfull * _CH, rem)])
        plsc.subcore_barrier()

        def wait_g(b):
            pltpu.make_async_copy(h_hbm.at[srcv.at[0]], rows[b], gs[b]).wait()

        def wait_s(b):
            pltpu.make_async_copy(rows[b], agg.at[dstv.at[0]], ss[b]).wait()

        nph = jnp.where(c == 0, _K0, _K1)
        base0 = jnp.where(c == 0, s * (_CPP * _K0),
                          _NS * _CPP * _K0 + s * (_CPP * _K1))
        for p in range(max(_K0, _K1)):
            @pl.when(p < nph)
            def _(p=p):
                base = pl.multiple_of(base0 + p * _CPP, 8)
                pltpu.sync_copy(src_hbm.at[pl.ds(base, _CPP)], srcv)
                pltpu.sync_copy(dst_hbm.at[pl.ds(base, _CPP)], dstv)
                for b in range(_NB):
                    pltpu.async_copy(h_hbm.at[srcv.at[b]], rows[b], gs[b])

                def step(j2, carry):
                    j0 = _NB * j2
                    for b in range(_NB):
                        wait_g(b)
                        pltpu.async_copy(rows[b], agg.at[dstv.at[j0 + b]],
                                         ss[b], add=True)
                    for b in range(_NB):
                        jn = j0 + b + _NB
                        wait_s(b)

                        @pl.when(jn < _CPP)
                        def _(jn=jn, b=b):
                            pltpu.async_copy(h_hbm.at[srcv.at[jn]], rows[b],
                                             gs[b])
                    return carry

                lax.fori_loop(0, _CPP // _NB, step, 0)
        plsc.subcore_barrier()
        pltpu.sync_copy(agg.at[pl.ds(r0, _RPT)], out_hbm.at[c, pl.ds(r0, _RPT)])

    return sc_agg


def _tc_mid(h, agg2, W1, b1, g, be, W2, b2):
    def body(h_ref, a_ref, w1, b1r, gr, ber, w2, b2r, o_ref):
        z = h_ref[...] + a_ref[0, :_N, :] + a_ref[1, :_N, :]
        t = jnp.dot(z, w1[...], preferred_element_type=jnp.float32) + b1r[...]
        mu = jnp.mean(t, axis=0, keepdims=True)
        var = jnp.mean(jnp.square(t - mu), axis=0, keepdims=True)
        t = (t - mu) / jnp.sqrt(var + _EPS) * gr[...] + ber[...]
        t = jnp.maximum(t, 0.0)
        o = jnp.dot(t, w2[...], preferred_element_type=jnp.float32) + b2r[...]
        o_ref[...] = jnp.maximum(o, 0.0)

    return pl.pallas_call(
        body, out_shape=jax.ShapeDtypeStruct((_N, _DH), jnp.float32),
    )(h, agg2, W1, b1.reshape(1, -1), g.reshape(1, -1), be.reshape(1, -1),
      W2, b2.reshape(1, -1))


def _tc_last(h, agg2, W1, b1, g, be, W2, b2, Wc, bc):
    d_out = Wc.shape[1]

    def body(h_ref, a_ref, w1, b1r, gr, ber, w2, b2r, wc, bcr, o_ref):
        z = h_ref[...] + a_ref[0, :_N, :] + a_ref[1, :_N, :]
        t = jnp.dot(z, w1[...], preferred_element_type=jnp.float32) + b1r[...]
        mu = jnp.mean(t, axis=0, keepdims=True)
        var = jnp.mean(jnp.square(t - mu), axis=0, keepdims=True)
        t = (t - mu) / jnp.sqrt(var + _EPS) * gr[...] + ber[...]
        t = jnp.maximum(t, 0.0)
        o = jnp.dot(t, w2[...], preferred_element_type=jnp.float32) + b2r[...]
        hh = jnp.maximum(o, 0.0)
        o_ref[...] = jnp.dot(hh, wc[...], preferred_element_type=jnp.float32) + bcr[...]

    return pl.pallas_call(
        body, out_shape=jax.ShapeDtypeStruct((_N, d_out), jnp.float32),
    )(h, agg2, W1, b1.reshape(1, -1), g.reshape(1, -1), be.reshape(1, -1),
      W2, b2.reshape(1, -1), Wc, bc.reshape(1, -1))


def kernel(x, edge_attr, edge_index,
           W1_0, b1_0, g_0, be_0, W2_0, b2_0,
           W1_1, b1_1, g_1, be_1, W2_1, b2_1,
           W1_2, b1_2, g_2, be_2, W2_2, b2_2,
           Wc, bc):
    del edge_attr
    src = edge_index[0]
    dst = edge_index[1]
    pad = _EPAD - _E
    src2d = jnp.concatenate(
        [src, jnp.zeros((pad,), jnp.int32)]).reshape(_EPAD // _CH, _CH)
    dst2d = jnp.concatenate(
        [dst, jnp.full((pad,), _N, jnp.int32)]).reshape(_EPAD // _CH, _CH)
    zeros = jnp.zeros((_CH, _DH), jnp.float32)

    sc_agg = _build_sc_agg()
    params = [
        (W1_0, b1_0, g_0, be_0, W2_0, b2_0),
        (W1_1, b1_1, g_1, be_1, W2_1, b2_1),
    ]
    h = x
    for (W1, b1, g, be, W2, b2) in params:
        agg2 = sc_agg(h, src2d, dst2d, zeros)
        h = _tc_mid(h, agg2, W1, b1, g, be, W2, b2)
    agg2 = sc_agg(h, src2d, dst2d, zeros)
    return _tc_last(h, agg2, W1_2, b1_2, g_2, be_2, W2_2, b2_2, Wc, bc)


# speedup vs baseline: 1.0133x; 1.0133x over previous
import functools

import jax
import jax.numpy as jnp
from jax import lax
from jax.experimental import pallas as pl
from jax.experimental.pallas import tpu as pltpu
from jax.experimental.pallas import tpu_sc as plsc

_N = 10000
_DH = 128
_E = 320000
_EPS = 1e-5

_NC = 2
_NS = 16
_NW = _NC * _NS
_CH = 64
_EPAD = 327680
_NROW_PAD = 10112
_RPT = _NROW_PAD // _NS
_CPP = 8
_K0 = 38
_K1 = 2
_NB = 4


def _build_sc_agg():
    mesh = plsc.VectorSubcoreMesh(core_axis_name="c", subcore_axis_name="s")

    @functools.partial(
        pl.kernel,
        out_type=jax.ShapeDtypeStruct((_NC, _NROW_PAD, _DH), jnp.float32),
        mesh=mesh,
        scratch_types=[
            [pltpu.VMEM((_CPP, _CH), jnp.int32) for _ in range(2)],
            [pltpu.VMEM((_CPP, _CH), jnp.int32) for _ in range(2)],
            [pltpu.VMEM((_CH, _DH), jnp.float32) for _ in range(_NB)],
            pltpu.VMEM_SHARED((_NROW_PAD, _DH), jnp.float32),
            [pltpu.SemaphoreType.DMA for _ in range(_NB)],
            [pltpu.SemaphoreType.DMA for _ in range(_NB)],
            [pltpu.SemaphoreType.DMA for _ in range(2)],
        ],
    )
    def sc_agg(h_hbm, src_hbm, dst_hbm, zeros_hbm, out_hbm,
               srcv, dstv, rows, agg, gs, ss, isem):
        c = lax.axis_index("c")
        s = lax.axis_index("s")
        w = c * _NS + s
        pltpu.sync_copy(zeros_hbm, rows[0])
        r0 = s * _RPT
        nfull = _RPT // _CH
        for zi in range(nfull):
            pltpu.sync_copy(rows[0], agg.at[pl.ds(r0 + zi * _CH, _CH)])
        rem = _RPT - nfull * _CH
        if rem:
            pltpu.sync_copy(rows[0].at[pl.ds(0, rem)],
                            agg.at[pl.ds(r0 + nfull * _CH, rem)])
        plsc.subcore_barrier()

        def wait_g(b):
            pltpu.make_async_copy(h_hbm.at[srcv[0].at[0]], rows[b],
                                  gs[b]).wait()

        def wait_s(b):
            pltpu.make_async_copy(rows[b], agg.at[dstv[0].at[0]],
                                  ss[b]).wait()

        def wait_i(pb):
            pltpu.make_async_copy(src_hbm.at[pl.ds(0, _CPP)], srcv[pb],
                                  isem[pb]).wait()
            pltpu.make_async_copy(dst_hbm.at[pl.ds(0, _CPP)], dstv[pb],
                                  isem[pb]).wait()

        nph = jnp.where(c == 0, _K0, _K1)
        base0 = jnp.where(c == 0, s * (_CPP * _K0),
                          _NS * _CPP * _K0 + s * (_CPP * _K1))

        def base(p):
            return pl.multiple_of(base0 + p * _CPP, 8)

        pltpu.sync_copy(src_hbm.at[pl.ds(base(0), _CPP)], srcv[0])
        pltpu.sync_copy(dst_hbm.at[pl.ds(base(0), _CPP)], dstv[0])
        for b in range(_NB):
            pltpu.async_copy(h_hbm.at[srcv[0].at[b]], rows[b], gs[b])

        for p in range(max(_K0, _K1)):
            pb = p % 2
            qb = (p + 1) % 2

            @pl.when(p < nph)
            def _(p=p, pb=pb, qb=qb):
                @pl.when(p + 1 < nph)
                def _(p=p, qb=qb):
                    pltpu.async_copy(src_hbm.at[pl.ds(base(p + 1), _CPP)],
                                     srcv[qb], isem[qb])
                    pltpu.async_copy(dst_hbm.at[pl.ds(base(p + 1), _CPP)],
                                     dstv[qb], isem[qb])

                for j in range(_CPP):
                    b = j % _NB
                    wait_g(b)
                    pltpu.async_copy(rows[b], agg.at[dstv[pb].at[j]],
                                     ss[b], add=True)
                    wait_s(b)
                    jn = j + _NB
                    if jn < _CPP:
                        pltpu.async_copy(h_hbm.at[srcv[pb].at[jn]], rows[b],
                                         gs[b])
                    else:
                        if j == _CPP - _NB:
                            @pl.when(p + 1 < nph)
                            def _(qb=qb):
                                wait_i(qb)

                        @pl.when(p + 1 < nph)
                        def _(jn=jn, b=b, qb=qb):
                            pltpu.async_copy(h_hbm.at[srcv[qb].at[jn - _CPP]],
                                             rows[b], gs[b])
        plsc.subcore_barrier()
        pltpu.sync_copy(agg.at[pl.ds(r0, _RPT)], out_hbm.at[c, pl.ds(r0, _RPT)])

    return sc_agg


def _tc_mid(h, agg2, W1, b1, g, be, W2, b2):
    def body(h_ref, a_ref, w1, b1r, gr, ber, w2, b2r, o_ref):
        z = h_ref[...] + a_ref[0, :_N, :] + a_ref[1, :_N, :]
        t = jnp.dot(z, w1[...], preferred_element_type=jnp.float32) + b1r[...]
        mu = jnp.mean(t, axis=0, keepdims=True)
        var = jnp.mean(jnp.square(t - mu), axis=0, keepdims=True)
        t = (t - mu) / jnp.sqrt(var + _EPS) * gr[...] + ber[...]
        t = jnp.maximum(t, 0.0)
        o = jnp.dot(t, w2[...], preferred_element_type=jnp.float32) + b2r[...]
        o_ref[...] = jnp.maximum(o, 0.0)

    return pl.pallas_call(
        body, out_shape=jax.ShapeDtypeStruct((_N, _DH), jnp.float32),
    )(h, agg2, W1, b1.reshape(1, -1), g.reshape(1, -1), be.reshape(1, -1),
      W2, b2.reshape(1, -1))


def _tc_last(h, agg2, W1, b1, g, be, W2, b2, Wc, bc):
    d_out = Wc.shape[1]

    def body(h_ref, a_ref, w1, b1r, gr, ber, w2, b2r, wc, bcr, o_ref):
        z = h_ref[...] + a_ref[0, :_N, :] + a_ref[1, :_N, :]
        t = jnp.dot(z, w1[...], preferred_element_type=jnp.float32) + b1r[...]
        mu = jnp.mean(t, axis=0, keepdims=True)
        var = jnp.mean(jnp.square(t - mu), axis=0, keepdims=True)
        t = (t - mu) / jnp.sqrt(var + _EPS) * gr[...] + ber[...]
        t = jnp.maximum(t, 0.0)
        o = jnp.dot(t, w2[...], preferred_element_type=jnp.float32) + b2r[...]
        hh = jnp.maximum(o, 0.0)
        o_ref[...] = jnp.dot(hh, wc[...], preferred_element_type=jnp.float32) + bcr[...]

    return pl.pallas_call(
        body, out_shape=jax.ShapeDtypeStruct((_N, d_out), jnp.float32),
    )(h, agg2, W1, b1.reshape(1, -1), g.reshape(1, -1), be.reshape(1, -1),
      W2, b2.reshape(1, -1), Wc, bc.reshape(1, -1))


def kernel(x, edge_attr, edge_index,
           W1_0, b1_0, g_0, be_0, W2_0, b2_0,
           W1_1, b1_1, g_1, be_1, W2_1, b2_1,
           W1_2, b1_2, g_2, be_2, W2_2, b2_2,
           Wc, bc):
    del edge_attr
    src = edge_index[0]
    dst = edge_index[1]
    pad = _EPAD - _E
    src2d = jnp.concatenate(
        [src, jnp.zeros((pad,), jnp.int32)]).reshape(_EPAD // _CH, _CH)
    dst2d = jnp.concatenate(
        [dst, jnp.full((pad,), _N, jnp.int32)]).reshape(_EPAD // _CH, _CH)
    zeros = jnp.zeros((_CH, _DH), jnp.float32)

    sc_agg = _build_sc_agg()
    params = [
        (W1_0, b1_0, g_0, be_0, W2_0, b2_0),
        (W1_1, b1_1, g_1, be_1, W2_1, b2_1),
    ]
    h = x
    for (W1, b1, g, be, W2, b2) in params:
        agg2 = sc_agg(h, src2d, dst2d, zeros)
        h = _tc_mid(h, agg2, W1, b1, g, be, W2, b2)
    agg2 = sc_agg(h, src2d, dst2d, zeros)
    return _tc_last(h, agg2, W1_2, b1_2, g_2, be_2, W2_2, b2_2, Wc, bc)
